# Initial kernel scaffold; baseline (speedup 1.0000x reference)
import jax
import jax.numpy as jnp
from jax.experimental import pallas as pl


def kernel(x):
    raise NotImplementedError("write your pallas kernel here")


# speedup vs baseline: 2.3775x; 2.3775x over previous
import jax
import jax.numpy as jnp
from jax.experimental import pallas as pl

_NUM_ANCHORS = 3
_NUM_CLASSES = 80
_NATTR = _NUM_CLASSES + 5
_IMG_SIZE = 416.0
_ANCH_W = (10.0, 16.0, 33.0)
_ANCH_H = (13.0, 30.0, 23.0)


def _decode_body(x_ref, o_ref, *, g, stride):
    cells = g * g
    a = pl.program_id(1)
    v = x_ref[...]

    r = jax.lax.broadcasted_iota(jnp.int32, (_NATTR, cells), 0)
    c = jax.lax.broadcasted_iota(jnp.int32, (_NATTR, cells), 1)

    sig = jax.nn.sigmoid(v)
    ex = jnp.exp(v)

    aw = jnp.where(a == 0, _ANCH_W[0], jnp.where(a == 1, _ANCH_W[1], _ANCH_W[2]))
    ah = jnp.where(a == 0, _ANCH_H[0], jnp.where(a == 1, _ANCH_H[1], _ANCH_H[2]))
    anch = jnp.where(r == 2, aw, ah).astype(jnp.float32)

    is_wh = (r == 2) | (r == 3)
    base = jnp.where(is_wh, ex * anch, sig)

    gx = (c % g).astype(jnp.float32)
    gy = (c // g).astype(jnp.float32)
    add = jnp.where(r == 0, gx, jnp.where(r == 1, gy, 0.0))
    scale = jnp.where(r <= 1, jnp.float32(stride), jnp.float32(1.0))

    res = (base + add) * scale
    o_ref[...] = res.T


def kernel(x):
    B = x.shape[0]
    g = x.shape[2]
    cells = g * g
    stride = _IMG_SIZE / g

    x4 = x.reshape(B, _NUM_ANCHORS, _NATTR, cells)

    out = pl.pallas_call(
        lambda x_ref, o_ref: _decode_body(x_ref, o_ref, g=g, stride=stride),
        grid=(B, _NUM_ANCHORS),
        in_specs=[
            pl.BlockSpec(
                (None, None, _NATTR, cells), lambda b, a: (b, a, 0, 0)
            )
        ],
        out_specs=pl.BlockSpec((None, cells, _NATTR), lambda b, a: (b, a, 0)),
        out_shape=jax.ShapeDtypeStruct(
            (B, _NUM_ANCHORS * cells, _NATTR), jnp.float32
        ),
    )(x4)
    return out
